# baseline (speedup 1.0000x reference)
import jax
import jax.numpy as jnp
from jax import lax
from jax.experimental import pallas as pl
from jax.experimental.pallas import tpu as pltpu
from jax.experimental.pallas import tpu_sc as plsc

N = 10000
NP = 10240
E = 160000
EP = 163840
NW = 32
B = 128
NB = EP // (NW * B)
RPT = NP // 16
BLK = 512
GRID = NP // BLK


def _sc_mesh():
    return plsc.VectorSubcoreMesh(
        core_axis_name="c", subcore_axis_name="s", num_cores=2, num_subcores=16
    )


def _hist_body(dst_hbm, out_hbm, idx_d, ones_v, acc):
    cid = lax.axis_index("c")
    sid = lax.axis_index("s")
    wid = cid * 16 + sid
    r0 = sid * RPT

    def fill(val):
        vv = jnp.full((16,), val, jnp.float32)

        def frow(i, carry):
            ones_v[i, pl.ds(0, 16)] = vv
            return carry

        lax.fori_loop(0, B, frow, 0)

    fill(0.0)
    for t in range(RPT // B):
        pltpu.sync_copy(ones_v, acc.at[pl.ds(r0 + t * B, B)])
    fill(1.0)
    pltpu.sync_copy(dst_hbm.at[pl.ds(wid * NB, NB)], idx_d)
    plsc.subcore_barrier()

    def step(j, carry):
        pltpu.sync_copy(ones_v, acc.at[idx_d.at[j]], add=True)
        return carry

    lax.fori_loop(0, NB, step, 0)
    plsc.subcore_barrier()
    pltpu.sync_copy(acc.at[pl.ds(r0, RPT)], out_hbm.at[wid])


_sc_hist = pl.kernel(
    _hist_body,
    out_type=jax.ShapeDtypeStruct((NW, RPT, 16), jnp.float32),
    mesh=_sc_mesh(),
    scratch_types=[
        pltpu.VMEM((NB, B), jnp.int32),
        pltpu.VMEM((B, 16), jnp.float32),
        pltpu.VMEM_SHARED((NP, 16), jnp.float32),
    ],
    compiler_params=pltpu.CompilerParams(use_tc_tiling_on_sc=False),
)


def _make_scatter(F, nbuf):
    assert NB % nbuf == 0

    def body(src_hbm, dst_hbm, y_hbm, out_hbm, idx_s, idx_d, *scratch):
        rows = list(scratch[:nbuf])
        y_sh = scratch[nbuf]
        acc = scratch[nbuf + 1]
        gsem = list(scratch[nbuf + 2:2 * nbuf + 2])
        ssem = list(scratch[2 * nbuf + 2:])
        cid = lax.axis_index("c")
        sid = lax.axis_index("s")
        wid = cid * 16 + sid
        r0 = sid * RPT

        zv = jnp.zeros((16,), jnp.float32)

        def zrow(i, carry):
            for k in range(F // 16):
                rows[0][i, pl.ds(k * 16, 16)] = zv
            return carry

        lax.fori_loop(0, B, zrow, 0)
        for t in range(RPT // B):
            pltpu.sync_copy(rows[0], acc.at[pl.ds(r0 + t * B, B)])
        pltpu.sync_copy(y_hbm.at[pl.ds(r0, RPT)], y_sh.at[pl.ds(r0, RPT)])
        pltpu.sync_copy(src_hbm.at[pl.ds(wid * NB, NB)], idx_s)
        pltpu.sync_copy(dst_hbm.at[pl.ds(wid * NB, NB)], idx_d)
        plsc.subcore_barrier()

        def group(gi, carry):
            descs = []
            for b in range(nbuf):
                j = gi * nbuf + b

                @pl.when(gi > 0)
                def _(b=b, j=j):
                    pltpu.make_async_copy(
                        rows[b], acc.at[idx_d.at[j]], ssem[b]
                    ).wait()

                descs.append(
                    pltpu.async_copy(y_sh.at[idx_s.at[j]], rows[b], gsem[b])
                )
            for b in range(nbuf):
                j = gi * nbuf + b
                descs[b].wait()
                pltpu.async_copy(rows[b], acc.at[idx_d.at[j]], ssem[b],
                                 add=True)
            return carry

        lax.fori_loop(0, NB // nbuf, group, 0)
        for b in range(nbuf):
            pltpu.make_async_copy(rows[b], acc.at[idx_d.at[b]], ssem[b]).wait()
        plsc.subcore_barrier()
        pltpu.sync_copy(acc.at[pl.ds(r0, RPT)],
                        out_hbm.at[cid * 16 + sid])

    return pl.kernel(
        body,
        out_type=jax.ShapeDtypeStruct((NW, RPT, F), jnp.float32),
        mesh=_sc_mesh(),
        scratch_types=[
            pltpu.VMEM((NB, B), jnp.int32),
            pltpu.VMEM((NB, B), jnp.int32),
        ] + [pltpu.VMEM((B, F), jnp.float32)] * nbuf + [
            pltpu.VMEM_SHARED((NP, F), jnp.float32),
            pltpu.VMEM_SHARED((NP, F), jnp.float32),
        ] + [pltpu.SemaphoreType.DMA] * (2 * nbuf),
        compiler_params=pltpu.CompilerParams(use_tc_tiling_on_sc=False),
    )


_sc_scat32 = _make_scatter(32, 8)
_sc_scat64 = _make_scatter(64, 4)


RBLK = RPT
RGRID = NP // RBLK


def _full(shape):
    return pl.BlockSpec(shape, lambda i: tuple(0 for _ in shape))


def _raw_specs(F):
    return [
        pl.BlockSpec((1, RBLK, F), lambda i: (i, 0, 0)),
        pl.BlockSpec((1, RBLK, F), lambda i: (16 + i, 0, 0)),
    ]


def _sum_partials(raw_ref, v0, v1, sem0, sem1):
    i = pl.program_id(0)
    c0 = pltpu.async_copy(raw_ref.at[i], v0, sem0)
    c1 = pltpu.async_copy(raw_ref.at[16 + i], v1, sem1)
    c0.wait()
    c1.wait()
    return v0[...] + v1[...]


def _rowmask(val):
    row = pl.program_id(0) * RBLK + lax.broadcasted_iota(
        jnp.int32, (RBLK, 1), 0
    )
    return jnp.where(row < N, val, 0.0)


def _raw_scratch(F):
    return (
        [pltpu.VMEM((RBLK, F), jnp.float32)] * 2
        + [pltpu.SemaphoreType.DMA] * 2
    )


def _prep1_body(h_ref, x_ref, w_ref, dinv_ref, pre_ref, v0, v1, s0, s1):
    hsum = _sum_partials(h_ref, v0, v1, s0, s1)
    deg = hsum[:, 0:1] + 1.0
    dinv = lax.rsqrt(deg)
    dinv_ref[...] = dinv
    xw = jnp.dot(x_ref[...], w_ref[...], preferred_element_type=jnp.float32)
    pre_ref[...] = xw * dinv


_tc_prep1 = pl.pallas_call(
    _prep1_body,
    grid=(RGRID,),
    in_specs=[
        pl.BlockSpec(memory_space=pl.ANY),
        pl.BlockSpec((RBLK, 128), lambda i: (i, 0)),
        _full((128, 64)),
    ],
    out_specs=[
        pl.BlockSpec((RBLK, 1), lambda i: (i, 0)),
        pl.BlockSpec((RBLK, 64), lambda i: (i, 0)),
    ],
    out_shape=[
        jax.ShapeDtypeStruct((NP, 1), jnp.float32),
        jax.ShapeDtypeStruct((NP, 64), jnp.float32),
    ],
    scratch_shapes=_raw_scratch(16),
)


def _post1_body(raw_ref, pre, dinv_r, b_r, w_r, pre2_o, v0, v1, s0, s1):
    rsum = _sum_partials(raw_ref, v0, v1, s0, s1)
    dinv = dinv_r[...]
    h = (rsum + pre[...]) * dinv + b_r[...]
    h = _rowmask(jnp.maximum(h, 0.0))
    pre2_o[...] = (
        jnp.dot(h, w_r[...], preferred_element_type=jnp.float32) * dinv
    )


_tc_post1 = pl.pallas_call(
    _post1_body,
    grid=(RGRID,),
    in_specs=[
        pl.BlockSpec(memory_space=pl.ANY),
        pl.BlockSpec((RBLK, 64), lambda i: (i, 0)),
        pl.BlockSpec((RBLK, 1), lambda i: (i, 0)),
        _full((1, 64)),
        _full((64, 64)),
    ],
    out_specs=pl.BlockSpec((RBLK, 64), lambda i: (i, 0)),
    out_shape=jax.ShapeDtypeStruct((NP, 64), jnp.float32),
    scratch_shapes=_raw_scratch(64),
)


def _post2_body(raw_ref, pre, dinv_r, b_r, out2_o, pre3_o, v0, v1, s0, s1):
    rsum = _sum_partials(raw_ref, v0, v1, s0, s1)
    dinv = dinv_r[...]
    out2 = _rowmask((rsum + pre[...]) * dinv + b_r[...])
    out2_o[...] = out2
    pre3_o[...] = out2[:, :32] * dinv


_tc_post2 = pl.pallas_call(
    _post2_body,
    grid=(RGRID,),
    in_specs=[
        pl.BlockSpec(memory_space=pl.ANY),
        pl.BlockSpec((RBLK, 64), lambda i: (i, 0)),
        pl.BlockSpec((RBLK, 1), lambda i: (i, 0)),
        _full((1, 64)),
    ],
    out_specs=[
        pl.BlockSpec((RBLK, 64), lambda i: (i, 0)),
        pl.BlockSpec((RBLK, 32), lambda i: (i, 0)),
    ],
    out_shape=[
        jax.ShapeDtypeStruct((NP, 64), jnp.float32),
        jax.ShapeDtypeStruct((NP, 32), jnp.float32),
    ],
    scratch_shapes=_raw_scratch(64),
)


def _post3_body(raw_ref, pre, dinv_r, b_r, w_r, pre4_o, v0, v1, s0, s1):
    rsum = _sum_partials(raw_ref, v0, v1, s0, s1)
    dinv = dinv_r[...]
    pz = (rsum + pre[...]) * dinv
    d = jnp.maximum(
        jnp.dot(pz, w_r[...], preferred_element_type=jnp.float32) + b_r[...],
        0.0,
    )
    pre4_o[...] = _rowmask(d) * dinv


_tc_post3 = pl.pallas_call(
    _post3_body,
    grid=(RGRID,),
    in_specs=[
        pl.BlockSpec(memory_space=pl.ANY),
        pl.BlockSpec((RBLK, 32), lambda i: (i, 0)),
        pl.BlockSpec((RBLK, 1), lambda i: (i, 0)),
        _full((1, 64)),
        _full((32, 64)),
    ],
    out_specs=pl.BlockSpec((RBLK, 64), lambda i: (i, 0)),
    out_shape=jax.ShapeDtypeStruct((NP, 64), jnp.float32),
    scratch_shapes=_raw_scratch(32),
)


def _post4_body(raw_ref, pre, dinv_r, b_r, w_r, xp_o, v0, v1, s0, s1):
    rsum = _sum_partials(raw_ref, v0, v1, s0, s1)
    dinv = dinv_r[...]
    pd = (rsum + pre[...]) * dinv
    xp_o[...] = jax.nn.sigmoid(
        jnp.dot(pd, w_r[...], preferred_element_type=jnp.float32) + b_r[...]
    )


_tc_post4 = pl.pallas_call(
    _post4_body,
    grid=(RGRID,),
    in_specs=[
        pl.BlockSpec(memory_space=pl.ANY),
        pl.BlockSpec((RBLK, 64), lambda i: (i, 0)),
        pl.BlockSpec((RBLK, 1), lambda i: (i, 0)),
        _full((1, 128)),
        _full((64, 128)),
    ],
    out_specs=pl.BlockSpec((RBLK, 128), lambda i: (i, 0)),
    out_shape=jax.ShapeDtypeStruct((N, 128), jnp.float32),
    scratch_shapes=_raw_scratch(64),
)


def _decode_body(zi_ref, zj_ref, out_ref):
    zi = zi_ref[:, :32]
    zj = zj_ref[:, :32]
    s = lax.dot_general(
        zi, zj, (((1,), (1,)), ((), ())), preferred_element_type=jnp.float32
    )
    out_ref[...] = jax.nn.sigmoid(s)


def _decode_stripe_body(zi_ref, zj_ref, prev_ref, out_ref):
    del prev_ref
    _decode_body(zi_ref, zj_ref, out_ref)


NSTRIPE = 4
SROWS = GRID // NSTRIPE


def _make_decode_stripe(k):
    in_specs = [
        pl.BlockSpec((BLK, 64), lambda i, j: (k * SROWS + i, 0)),
        pl.BlockSpec((BLK, 64), lambda i, j: (j, 0)),
    ]
    if k == 0:
        return pl.pallas_call(
            _decode_body,
            grid=(SROWS, GRID),
            in_specs=in_specs,
            out_specs=pl.BlockSpec((BLK, BLK),
                                   lambda i, j: (k * SROWS + i, j)),
            out_shape=jax.ShapeDtypeStruct((N, N), jnp.float32),
        )
    return pl.pallas_call(
        _decode_stripe_body,
        grid=(SROWS, GRID),
        in_specs=in_specs + [pl.BlockSpec(memory_space=pl.ANY)],
        out_specs=pl.BlockSpec((BLK, BLK), lambda i, j: (k * SROWS + i, j)),
        out_shape=jax.ShapeDtypeStruct((N, N), jnp.float32),
        input_output_aliases={2: 0},
    )


_tc_decode_stripes = [_make_decode_stripe(k) for k in range(NSTRIPE)]


def kernel(x_features, edge_index, W1, b1, Wmu, bmu, Wlv, blv, W2, b2, W3, b3):
    f32 = jnp.float32
    src = edge_index[0].astype(jnp.int32)
    dst = edge_index[1].astype(jnp.int32)
    padi = jnp.full((EP - E,), N, dtype=jnp.int32)
    src3 = jnp.concatenate([src, padi]).reshape(EP // B, B)
    dst3 = jnp.concatenate([dst, padi]).reshape(EP // B, B)

    x_pad = jnp.zeros((NP, 128), f32).at[:N].set(x_features)

    Wcat = jnp.concatenate([Wmu, Wlv], axis=1)
    bcat = jnp.concatenate([bmu, blv]).reshape(1, 64)

    hist = _sc_hist(dst3)
    dinv, pre1 = _tc_prep1(hist, x_pad, W1)

    raw1 = _sc_scat64(src3, dst3, pre1)
    pre2 = _tc_post1(raw1, pre1, dinv, b1.reshape(1, 64), Wcat)

    raw2 = _sc_scat64(src3, dst3, pre2)
    out2z, pre3 = _tc_post2(raw2, pre2, dinv, bcat)

    A_pred = _tc_decode_stripes[0](out2z, out2z)
    for stripe in _tc_decode_stripes[1:]:
        A_pred = stripe(out2z, out2z, A_pred)

    raw3 = _sc_scat32(src3, dst3, pre3)
    pre4 = _tc_post3(raw3, pre3, dinv, b2.reshape(1, 64), W2)

    raw4 = _sc_scat64(src3, dst3, pre4)
    x_pred = _tc_post4(raw4, pre4, dinv, b3.reshape(1, 128), W3)

    mu = out2z[:N, :32]
    logvar = out2z[:N, 32:]
    return A_pred, mu, logvar, mu, x_pred


# speedup vs baseline: 1.0968x; 1.0968x over previous
import jax
import jax.numpy as jnp
from jax import lax
from jax.experimental import pallas as pl
from jax.experimental.pallas import tpu as pltpu
from jax.experimental.pallas import tpu_sc as plsc

N = 10000
NP = 10240
E = 160000
EP = 163840
NW = 32
B = 128
NB = EP // (NW * B)
RPT = NP // 16
BLK = 512
GRID = NP // BLK


def _sc_mesh():
    return plsc.VectorSubcoreMesh(
        core_axis_name="c", subcore_axis_name="s", num_cores=2, num_subcores=16
    )


def _hist_body(dst_hbm, out_hbm, idx_d, ones_v, acc):
    cid = lax.axis_index("c")
    sid = lax.axis_index("s")
    wid = cid * 16 + sid
    r0 = sid * RPT

    def fill(val):
        vv = jnp.full((16,), val, jnp.float32)

        def frow(i, carry):
            ones_v[i, pl.ds(0, 16)] = vv
            return carry

        lax.fori_loop(0, B, frow, 0)

    fill(0.0)
    for t in range(RPT // B):
        pltpu.sync_copy(ones_v, acc.at[pl.ds(r0 + t * B, B)])
    fill(1.0)
    pltpu.sync_copy(dst_hbm.at[pl.ds(wid * NB, NB)], idx_d)
    plsc.subcore_barrier()

    def step(j, carry):
        pltpu.sync_copy(ones_v, acc.at[idx_d.at[j]], add=True)
        return carry

    lax.fori_loop(0, NB, step, 0)
    plsc.subcore_barrier()
    pltpu.sync_copy(acc.at[pl.ds(r0, RPT)], out_hbm.at[wid])


_sc_hist = pl.kernel(
    _hist_body,
    out_type=jax.ShapeDtypeStruct((NW, RPT, 16), jnp.float32),
    mesh=_sc_mesh(),
    scratch_types=[
        pltpu.VMEM((NB, B), jnp.int32),
        pltpu.VMEM((B, 16), jnp.float32),
        pltpu.VMEM_SHARED((NP, 16), jnp.float32),
    ],
    compiler_params=pltpu.CompilerParams(use_tc_tiling_on_sc=False),
)


def _make_scatter(F, nbuf):
    assert NB % nbuf == 0

    def body(src_hbm, dst_hbm, y_hbm, out_hbm, idx_s, idx_d, *scratch):
        rows = list(scratch[:nbuf])
        y_sh = scratch[nbuf]
        acc = scratch[nbuf + 1]
        gsem = list(scratch[nbuf + 2:2 * nbuf + 2])
        ssem = list(scratch[2 * nbuf + 2:])
        cid = lax.axis_index("c")
        sid = lax.axis_index("s")
        wid = cid * 16 + sid
        r0 = sid * RPT

        zv = jnp.zeros((16,), jnp.float32)

        def zrow(i, carry):
            for k in range(F // 16):
                rows[0][i, pl.ds(k * 16, 16)] = zv
            return carry

        lax.fori_loop(0, B, zrow, 0)
        for t in range(RPT // B):
            pltpu.sync_copy(rows[0], acc.at[pl.ds(r0 + t * B, B)])
        pltpu.sync_copy(y_hbm.at[pl.ds(r0, RPT)], y_sh.at[pl.ds(r0, RPT)])
        pltpu.sync_copy(src_hbm.at[pl.ds(wid * NB, NB)], idx_s)
        pltpu.sync_copy(dst_hbm.at[pl.ds(wid * NB, NB)], idx_d)
        plsc.subcore_barrier()

        def group(gi, carry):
            descs = []
            for b in range(nbuf):
                j = gi * nbuf + b

                @pl.when(gi > 0)
                def _(b=b, j=j):
                    pltpu.make_async_copy(
                        rows[b], acc.at[idx_d.at[j]], ssem[b]
                    ).wait()

                descs.append(
                    pltpu.async_copy(y_sh.at[idx_s.at[j]], rows[b], gsem[b])
                )
            for b in range(nbuf):
                j = gi * nbuf + b
                descs[b].wait()
                pltpu.async_copy(rows[b], acc.at[idx_d.at[j]], ssem[b],
                                 add=True)
            return carry

        lax.fori_loop(0, NB // nbuf, group, 0)
        for b in range(nbuf):
            pltpu.make_async_copy(rows[b], acc.at[idx_d.at[b]], ssem[b]).wait()
        plsc.subcore_barrier()
        pltpu.sync_copy(acc.at[pl.ds(r0, RPT)],
                        out_hbm.at[cid * 16 + sid])

    return pl.kernel(
        body,
        out_type=jax.ShapeDtypeStruct((NW, RPT, F), jnp.float32),
        mesh=_sc_mesh(),
        scratch_types=[
            pltpu.VMEM((NB, B), jnp.int32),
            pltpu.VMEM((NB, B), jnp.int32),
        ] + [pltpu.VMEM((B, F), jnp.float32)] * nbuf + [
            pltpu.VMEM_SHARED((NP, F), jnp.float32),
            pltpu.VMEM_SHARED((NP, F), jnp.float32),
        ] + [pltpu.SemaphoreType.DMA] * (2 * nbuf),
        compiler_params=pltpu.CompilerParams(use_tc_tiling_on_sc=False),
    )


_sc_scat32 = _make_scatter(32, 8)
_sc_scat64 = _make_scatter(64, 4)


RBLK = RPT
RGRID = NP // RBLK


def _full(shape):
    return pl.BlockSpec(shape, lambda i: tuple(0 for _ in shape))


def _raw_specs(F):
    return [
        pl.BlockSpec((1, RBLK, F), lambda i: (i, 0, 0)),
        pl.BlockSpec((1, RBLK, F), lambda i: (16 + i, 0, 0)),
    ]


def _rowmask(val):
    row = pl.program_id(0) * RBLK + lax.broadcasted_iota(
        jnp.int32, (RBLK, 1), 0
    )
    return jnp.where(row < N, val, 0.0)


def _prep1_body(h0_ref, h1_ref, x_ref, w_ref, dinv_ref, pre_ref):
    deg = h0_ref[0, :, 0:1] + h1_ref[0, :, 0:1] + 1.0
    dinv = lax.rsqrt(deg)
    dinv_ref[...] = dinv
    xw = jnp.dot(x_ref[...], w_ref[...], preferred_element_type=jnp.float32)
    pre_ref[...] = xw * dinv


_tc_prep1 = pl.pallas_call(
    _prep1_body,
    grid=(RGRID,),
    in_specs=_raw_specs(16) + [
        pl.BlockSpec((RBLK, 128), lambda i: (i, 0)),
        _full((128, 64)),
    ],
    out_specs=[
        pl.BlockSpec((RBLK, 1), lambda i: (i, 0)),
        pl.BlockSpec((RBLK, 64), lambda i: (i, 0)),
    ],
    out_shape=[
        jax.ShapeDtypeStruct((NP, 1), jnp.float32),
        jax.ShapeDtypeStruct((NP, 64), jnp.float32),
    ],
)


def _post1_body(r0, r1, pre, dinv_r, b_r, w_r, pre2_o):
    dinv = dinv_r[...]
    h = (r0[0] + r1[0] + pre[...]) * dinv + b_r[...]
    h = _rowmask(jnp.maximum(h, 0.0))
    pre2_o[...] = (
        jnp.dot(h, w_r[...], preferred_element_type=jnp.float32) * dinv
    )


_tc_post1 = pl.pallas_call(
    _post1_body,
    grid=(RGRID,),
    in_specs=_raw_specs(64) + [
        pl.BlockSpec((RBLK, 64), lambda i: (i, 0)),
        pl.BlockSpec((RBLK, 1), lambda i: (i, 0)),
        _full((1, 64)),
        _full((64, 64)),
    ],
    out_specs=pl.BlockSpec((RBLK, 64), lambda i: (i, 0)),
    out_shape=jax.ShapeDtypeStruct((NP, 64), jnp.float32),
)


def _post2_body(r0, r1, pre, dinv_r, b_r, out2_o, pre3_o):
    dinv = dinv_r[...]
    out2 = _rowmask((r0[0] + r1[0] + pre[...]) * dinv + b_r[...])
    out2_o[...] = out2
    pre3_o[...] = out2[:, :32] * dinv


_tc_post2 = pl.pallas_call(
    _post2_body,
    grid=(RGRID,),
    in_specs=_raw_specs(64) + [
        pl.BlockSpec((RBLK, 64), lambda i: (i, 0)),
        pl.BlockSpec((RBLK, 1), lambda i: (i, 0)),
        _full((1, 64)),
    ],
    out_specs=[
        pl.BlockSpec((RBLK, 64), lambda i: (i, 0)),
        pl.BlockSpec((RBLK, 32), lambda i: (i, 0)),
    ],
    out_shape=[
        jax.ShapeDtypeStruct((NP, 64), jnp.float32),
        jax.ShapeDtypeStruct((NP, 32), jnp.float32),
    ],
)


def _post3_body(r0, r1, pre, dinv_r, b_r, w_r, pre4_o):
    dinv = dinv_r[...]
    pz = (r0[0] + r1[0] + pre[...]) * dinv
    d = jnp.maximum(
        jnp.dot(pz, w_r[...], preferred_element_type=jnp.float32) + b_r[...],
        0.0,
    )
    pre4_o[...] = _rowmask(d) * dinv


_tc_post3 = pl.pallas_call(
    _post3_body,
    grid=(RGRID,),
    in_specs=_raw_specs(32) + [
        pl.BlockSpec((RBLK, 32), lambda i: (i, 0)),
        pl.BlockSpec((RBLK, 1), lambda i: (i, 0)),
        _full((1, 64)),
        _full((32, 64)),
    ],
    out_specs=pl.BlockSpec((RBLK, 64), lambda i: (i, 0)),
    out_shape=jax.ShapeDtypeStruct((NP, 64), jnp.float32),
)


def _post4_body(r0, r1, pre, dinv_r, b_r, w_r, xp_o):
    dinv = dinv_r[...]
    pd = (r0[0] + r1[0] + pre[...]) * dinv
    xp_o[...] = jax.nn.sigmoid(
        jnp.dot(pd, w_r[...], preferred_element_type=jnp.float32) + b_r[...]
    )


_tc_post4 = pl.pallas_call(
    _post4_body,
    grid=(RGRID,),
    in_specs=_raw_specs(64) + [
        pl.BlockSpec((RBLK, 64), lambda i: (i, 0)),
        pl.BlockSpec((RBLK, 1), lambda i: (i, 0)),
        _full((1, 128)),
        _full((64, 128)),
    ],
    out_specs=pl.BlockSpec((RBLK, 128), lambda i: (i, 0)),
    out_shape=jax.ShapeDtypeStruct((N, 128), jnp.float32),
)


def _decode_body(zi_ref, zj_ref, out_ref):
    zi = zi_ref[:, :32]
    zj = zj_ref[:, :32]
    s = lax.dot_general(
        zi, zj, (((1,), (1,)), ((), ())), preferred_element_type=jnp.float32
    )
    out_ref[...] = jax.nn.sigmoid(s)


def _decode_stripe_body(zi_ref, zj_ref, prev_ref, out_ref):
    del prev_ref
    _decode_body(zi_ref, zj_ref, out_ref)


NSTRIPE = 4
SROWS = GRID // NSTRIPE


def _make_decode_stripe(k):
    in_specs = [
        pl.BlockSpec((BLK, 64), lambda i, j: (k * SROWS + i, 0)),
        pl.BlockSpec((BLK, 64), lambda i, j: (j, 0)),
    ]
    if k == 0:
        return pl.pallas_call(
            _decode_body,
            grid=(SROWS, GRID),
            in_specs=in_specs,
            out_specs=pl.BlockSpec((BLK, BLK),
                                   lambda i, j: (k * SROWS + i, j)),
            out_shape=jax.ShapeDtypeStruct((N, N), jnp.float32),
        )
    return pl.pallas_call(
        _decode_stripe_body,
        grid=(SROWS, GRID),
        in_specs=in_specs + [pl.BlockSpec(memory_space=pl.ANY)],
        out_specs=pl.BlockSpec((BLK, BLK), lambda i, j: (k * SROWS + i, j)),
        out_shape=jax.ShapeDtypeStruct((N, N), jnp.float32),
        input_output_aliases={2: 0},
    )


_tc_decode_stripes = [_make_decode_stripe(k) for k in range(NSTRIPE)]


def kernel(x_features, edge_index, W1, b1, Wmu, bmu, Wlv, blv, W2, b2, W3, b3):
    f32 = jnp.float32
    src = edge_index[0].astype(jnp.int32)
    dst = edge_index[1].astype(jnp.int32)
    padi = jnp.full((EP - E,), N, dtype=jnp.int32)
    src3 = jnp.concatenate([src, padi]).reshape(EP // B, B)
    dst3 = jnp.concatenate([dst, padi]).reshape(EP // B, B)

    x_pad = jnp.zeros((NP, 128), f32).at[:N].set(x_features)

    Wcat = jnp.concatenate([Wmu, Wlv], axis=1)
    bcat = jnp.concatenate([bmu, blv]).reshape(1, 64)

    hist = _sc_hist(dst3)
    dinv, pre1 = _tc_prep1(hist, hist, x_pad, W1)

    raw1 = _sc_scat64(src3, dst3, pre1)
    pre2 = _tc_post1(raw1, raw1, pre1, dinv, b1.reshape(1, 64), Wcat)

    raw2 = _sc_scat64(src3, dst3, pre2)
    out2z, pre3 = _tc_post2(raw2, raw2, pre2, dinv, bcat)

    A_pred = _tc_decode_stripes[0](out2z, out2z)
    for stripe in _tc_decode_stripes[1:]:
        A_pred = stripe(out2z, out2z, A_pred)

    raw3 = _sc_scat32(src3, dst3, pre3)
    pre4 = _tc_post3(raw3, raw3, pre3, dinv, b2.reshape(1, 64), W2)

    raw4 = _sc_scat64(src3, dst3, pre4)
    x_pred = _tc_post4(raw4, raw4, pre4, dinv, b3.reshape(1, 128), W3)

    mu = out2z[:N, :32]
    logvar = out2z[:N, 32:]
    return A_pred, mu, logvar, mu, x_pred
